# baseline (speedup 1.0000x reference)
import functools

import jax
import jax.numpy as jnp
from jax import lax
from jax.experimental import pallas as pl
from jax.experimental.pallas import tpu as pltpu
from jax.experimental.pallas import tpu_sc as plsc

N = 10000
D = 128
NC = 2
NS = 16
LANES = 128
G = 4

N_PAD = 10240
TRASH = N
ZROWS = N_PAD // NS

def _deg_body(dst_hbm, ones_hbm, zeros_hbm, deg_out, idx_v, ones_v, z_v, acc):
    cid = lax.axis_index("c")
    sid = lax.axis_index("s")
    n_rows = dst_hbm.shape[1] // NS
    pltpu.sync_copy(ones_hbm, ones_v)
    pltpu.sync_copy(zeros_hbm, z_v)
    pltpu.sync_copy(z_v, acc.at[pl.ds(sid * ZROWS, ZROWS)])
    plsc.subcore_barrier()
    base = sid * n_rows

    def chunk(g, carry):
        pltpu.sync_copy(dst_hbm.at[cid, base + g], idx_v)
        pltpu.sync_copy(ones_v, acc.at[idx_v], add=True)
        return carry

    lax.fori_loop(0, n_rows, chunk, 0)
    plsc.subcore_barrier()
    pltpu.sync_copy(acc.at[pl.ds(sid * ZROWS, ZROWS)],
                    deg_out.at[cid, pl.ds(sid * ZROWS, ZROWS)])


HD = D // 2
NB = 8


def _spmm_body(src_hbm, dst_hbm, uv0_hbm, uv1_hbm, zeros_hbm, agg_out,
               sidx_l, didx_l, rows_l, z_v, acc, isem, gsem_l, ssem_l):
    cid = lax.axis_index("c")
    sid = lax.axis_index("s")
    n_rows = src_hbm.shape[1] // NS
    base = sid * n_rows
    pltpu.sync_copy(zeros_hbm, z_v)

    n_supers = n_rows // NB
    for p, uv_hbm in enumerate((uv0_hbm, uv1_hbm)):
        for t in range(ZROWS // LANES):
            pltpu.sync_copy(z_v, acc.at[pl.ds(sid * ZROWS + t * LANES, LANES)])
        plsc.subcore_barrier()

        for b in range(NB):
            pltpu.sync_copy(src_hbm.at[cid, base + b], sidx_l[b])
            pltpu.sync_copy(dst_hbm.at[cid, base + b], didx_l[b])
            pltpu.async_copy(uv_hbm.at[sidx_l[b]], rows_l[b], gsem_l[b])

        def chunk(s, carry):
            g1 = base + jnp.minimum(s + 1, n_supers - 1) * NB
            scps = []
            for b in range(NB):
                pltpu.make_async_copy(uv_hbm.at[sidx_l[b]], rows_l[b],
                                      gsem_l[b]).wait()
                scps.append(pltpu.async_copy(rows_l[b], acc.at[didx_l[b]],
                                             ssem_l[b], add=True))
            for b in range(NB):
                scps[b].wait()
                icp0 = pltpu.async_copy(src_hbm.at[cid, g1 + b], sidx_l[b], isem)
                icp1 = pltpu.async_copy(dst_hbm.at[cid, g1 + b], didx_l[b], isem)
                icp0.wait()
                icp1.wait()
                pltpu.async_copy(uv_hbm.at[sidx_l[b]], rows_l[b], gsem_l[b])
            return carry

        lax.fori_loop(0, n_supers, chunk, 0)
        for b in range(NB):
            pltpu.make_async_copy(uv_hbm.at[sidx_l[b]], rows_l[b],
                                  gsem_l[b]).wait()
        plsc.subcore_barrier()
        pltpu.sync_copy(acc.at[pl.ds(sid * ZROWS, ZROWS)],
                        agg_out.at[p, cid, pl.ds(sid * ZROWS, ZROWS)])


@functools.cache
def _sc_calls():
    mesh = plsc.VectorSubcoreMesh(core_axis_name="c", subcore_axis_name="s")
    deg_call = pl.kernel(
        _deg_body,
        out_type=jax.ShapeDtypeStruct((NC, N_PAD, 16), jnp.float32),
        mesh=mesh,
        scratch_types=[
            pltpu.VMEM((LANES,), jnp.int32),
            pltpu.VMEM((LANES, 16), jnp.float32),
            pltpu.VMEM((ZROWS, 16), jnp.float32),
            pltpu.VMEM_SHARED((N_PAD, 16), jnp.float32),
        ],
        compiler_params=pltpu.CompilerParams(use_tc_tiling_on_sc=False),
    )
    spmm_call = pl.kernel(
        _spmm_body,
        out_type=jax.ShapeDtypeStruct((2, NC, N_PAD, HD), jnp.float32),
        mesh=mesh,
        scratch_types=[
            [pltpu.VMEM((LANES,), jnp.int32) for _ in range(NB)],
            [pltpu.VMEM((LANES,), jnp.int32) for _ in range(NB)],
            [pltpu.VMEM((LANES, HD), jnp.float32) for _ in range(NB)],
            pltpu.VMEM((LANES, HD), jnp.float32),
            pltpu.VMEM_SHARED((N_PAD, HD), jnp.float32),
            pltpu.SemaphoreType.DMA,
            [pltpu.SemaphoreType.DMA for _ in range(NB)],
            [pltpu.SemaphoreType.DMA for _ in range(NB)],
        ],
        compiler_params=pltpu.CompilerParams(use_tc_tiling_on_sc=False),
    )
    return deg_call, spmm_call


def _gelu(x):
    return x * 0.5 * (1.0 + lax.erf(x * 0.7071067811865476))


def _scale(deg):
    return jnp.where(deg > 0, lax.rsqrt(deg), 0.0)[:, 0:1]


def _pre_body(h_ref, ideg_ref, odeg_ref, wi_ref, bi_ref, wo_ref, bo_ref,
              u_ref, v_ref):
    h = h_ref[...]
    s_src = _scale(odeg_ref[...])
    s_dst = _scale(ideg_ref[...])
    h1 = jnp.dot(h, wi_ref[...], preferred_element_type=jnp.float32) + bi_ref[...]
    h2 = jnp.dot(h, wo_ref[...], preferred_element_type=jnp.float32) + bo_ref[...]
    u_ref[...] = s_src * h1
    v_ref[...] = s_dst * h2


def _post_body(a0_ref, a1_ref, ideg_ref, odeg_ref, wf1_ref, bf1_ref,
               wf2_ref, bf2_ref, o_ref):
    s_dst = _scale(ideg_ref[...])
    s_src = _scale(odeg_ref[...])
    x_in = _gelu(s_dst * a0_ref[...])
    x_out = _gelu(s_src * a1_ref[...])
    cat = jnp.concatenate([x_in, x_out], axis=1)
    z = _gelu(jnp.dot(cat, wf1_ref[...], preferred_element_type=jnp.float32)
              + bf1_ref[...])
    o_ref[...] = (jnp.dot(z, wf2_ref[...], preferred_element_type=jnp.float32)
                  + bf2_ref[...])


BN = 1000


def _row_spec(w):
    return pl.BlockSpec((BN, w), lambda i: (i, 0))


def _full_spec(r, c):
    return pl.BlockSpec((r, c), lambda i: (0, 0))


_pre_call = pl.pallas_call(
    _pre_body,
    grid=(N // BN,),
    in_specs=[
        _row_spec(D), _row_spec(16), _row_spec(16),
        _full_spec(D, D), _full_spec(1, D),
        _full_spec(D, D), _full_spec(1, D),
    ],
    out_specs=[_row_spec(D), _row_spec(D)],
    out_shape=[
        jax.ShapeDtypeStruct((N, D), jnp.float32),
        jax.ShapeDtypeStruct((N, D), jnp.float32),
    ],
)

_post_call = pl.pallas_call(
    _post_body,
    grid=(N // BN,),
    in_specs=[
        _row_spec(D), _row_spec(D), _row_spec(16), _row_spec(16),
        _full_spec(2 * D, D), _full_spec(1, D),
        _full_spec(D, D), _full_spec(1, D),
    ],
    out_specs=_row_spec(D),
    out_shape=jax.ShapeDtypeStruct((N, D), jnp.float32),
)


def kernel(x, edge_index, W_in0, b_in0, W_out0, b_out0, Wf1_0, bf1_0,
           Wf2_0, bf2_0, W_in1, b_in1, W_out1, b_out1, Wf1_1, bf1_1,
           Wf2_1, bf2_1):
    E = edge_index.shape[1]
    blk = NS * LANES * G
    e_pad = -(-E // blk) * blk
    pad = e_pad - E

    row = edge_index[0]
    col = edge_index[1]
    src_p = jnp.concatenate(
        [jnp.stack([row, col + N]),
         jnp.zeros((NC, pad), jnp.int32)], axis=1).reshape(NC, e_pad // LANES, LANES)
    dst_p = jnp.concatenate(
        [jnp.stack([col, row]),
         jnp.full((NC, pad), TRASH, jnp.int32)], axis=1).reshape(NC, e_pad // LANES, LANES)

    ones16 = jnp.ones((LANES, 16), jnp.float32)
    zeros16 = jnp.zeros((ZROWS, 16), jnp.float32)
    zerosD = jnp.zeros((LANES, HD), jnp.float32)

    deg_call, spmm_call = _sc_calls()
    degs = deg_call(dst_p, ones16, zeros16)
    ideg = degs[0, :N]
    odeg = degs[1, :N]

    params = [
        (W_in0, b_in0, W_out0, b_out0, Wf1_0, bf1_0, Wf2_0, bf2_0),
        (W_in1, b_in1, W_out1, b_out1, Wf1_1, bf1_1, Wf2_1, bf2_1),
    ]
    h = x
    for (Wi, bi, Wo, bo, Wf1, bf1, Wf2, bf2) in params:
        u, v = _pre_call(h, ideg, odeg, Wi.T, bi.reshape(1, D),
                         Wo.T, bo.reshape(1, D))
        uv = jnp.concatenate([u, v], axis=0)
        agg = spmm_call(src_p, dst_p, uv[:, :HD], uv[:, HD:], zerosD)
        a0 = jnp.concatenate([agg[0, 0, :N], agg[1, 0, :N]], axis=1)
        a1 = jnp.concatenate([agg[0, 1, :N], agg[1, 1, :N]], axis=1)
        h = _post_call(a0, a1, ideg, odeg, Wf1.T, bf1.reshape(1, D),
                       Wf2.T, bf2.reshape(1, D))
    return h


# speedup vs baseline: 2.7557x; 2.7557x over previous
import functools

import jax
import jax.numpy as jnp
from jax import lax
from jax.experimental import pallas as pl
from jax.experimental.pallas import tpu as pltpu
from jax.experimental.pallas import tpu_sc as plsc

N = 10000
D = 128
NC = 2
NS = 16
LANES = 128
G = 4

N_PAD = 10240
TRASH = N
ZROWS = N_PAD // NS

def _deg_body(dst_hbm, ones_hbm, zeros_hbm, deg_out, idx_v, ones_v, z_v, acc):
    cid = lax.axis_index("c")
    sid = lax.axis_index("s")
    n_rows = dst_hbm.shape[1] // NS
    pltpu.sync_copy(ones_hbm, ones_v)
    pltpu.sync_copy(zeros_hbm, z_v)
    pltpu.sync_copy(z_v, acc.at[pl.ds(sid * ZROWS, ZROWS)])
    plsc.subcore_barrier()
    base = sid * n_rows

    def chunk(g, carry):
        pltpu.sync_copy(dst_hbm.at[cid, base + g], idx_v)
        pltpu.sync_copy(ones_v, acc.at[idx_v], add=True)
        return carry

    lax.fori_loop(0, n_rows, chunk, 0)
    plsc.subcore_barrier()
    pltpu.sync_copy(acc.at[pl.ds(sid * ZROWS, ZROWS)],
                    deg_out.at[cid, pl.ds(sid * ZROWS, ZROWS)])


HD = D // 2
NB = 8


def _vcopy_row(big, b, small):
    for w in range(LANES // 16):
        small[pl.ds(w * 16, 16)] = big[b, pl.ds(w * 16, 16)]


def _spmm_body(src_hbm, dst_hbm, uv0_hbm, uv1_hbm, zeros_hbm, agg_out,
               sidx_l, didx_l, rows_l, sbig, dbig, z_v, acc, isem,
               gsem_l, ssem_l):
    cid = lax.axis_index("c")
    sid = lax.axis_index("s")
    n_rows = src_hbm.shape[1] // NS
    base = sid * n_rows
    pltpu.sync_copy(zeros_hbm, z_v)

    n_supers = n_rows // NB
    for p, uv_hbm in enumerate((uv0_hbm, uv1_hbm)):
        for t in range(ZROWS // LANES):
            pltpu.sync_copy(z_v, acc.at[pl.ds(sid * ZROWS + t * LANES, LANES)])
        plsc.subcore_barrier()

        pltpu.sync_copy(src_hbm.at[cid, pl.ds(base, NB)], sbig)
        pltpu.sync_copy(dst_hbm.at[cid, pl.ds(base, NB)], dbig)
        for b in range(NB):
            _vcopy_row(sbig, b, sidx_l[b])
            _vcopy_row(dbig, b, didx_l[b])
            pltpu.async_copy(uv_hbm.at[sidx_l[b]], rows_l[b], gsem_l[b])

        def chunk(s, carry):
            g1 = base + jnp.minimum(s + 1, n_supers - 1) * NB
            icp0 = pltpu.async_copy(src_hbm.at[cid, pl.ds(g1, NB)], sbig, isem)
            icp1 = pltpu.async_copy(dst_hbm.at[cid, pl.ds(g1, NB)], dbig, isem)
            scps = []
            for b in range(NB):
                pltpu.make_async_copy(uv_hbm.at[sidx_l[b]], rows_l[b],
                                      gsem_l[b]).wait()
                scps.append(pltpu.async_copy(rows_l[b], acc.at[didx_l[b]],
                                             ssem_l[b], add=True))
            icp0.wait()
            icp1.wait()
            for b in range(NB):
                scps[b].wait()
                _vcopy_row(sbig, b, sidx_l[b])
                _vcopy_row(dbig, b, didx_l[b])
                pltpu.async_copy(uv_hbm.at[sidx_l[b]], rows_l[b], gsem_l[b])
            return carry

        lax.fori_loop(0, n_supers, chunk, 0)
        for b in range(NB):
            pltpu.make_async_copy(uv_hbm.at[sidx_l[b]], rows_l[b],
                                  gsem_l[b]).wait()
        plsc.subcore_barrier()
        pltpu.sync_copy(acc.at[pl.ds(sid * ZROWS, ZROWS)],
                        agg_out.at[p, cid, pl.ds(sid * ZROWS, ZROWS)])


@functools.cache
def _sc_calls():
    mesh = plsc.VectorSubcoreMesh(core_axis_name="c", subcore_axis_name="s")
    deg_call = pl.kernel(
        _deg_body,
        out_type=jax.ShapeDtypeStruct((NC, N_PAD, 16), jnp.float32),
        mesh=mesh,
        scratch_types=[
            pltpu.VMEM((LANES,), jnp.int32),
            pltpu.VMEM((LANES, 16), jnp.float32),
            pltpu.VMEM((ZROWS, 16), jnp.float32),
            pltpu.VMEM_SHARED((N_PAD, 16), jnp.float32),
        ],
        compiler_params=pltpu.CompilerParams(use_tc_tiling_on_sc=False),
    )
    spmm_call = pl.kernel(
        _spmm_body,
        out_type=jax.ShapeDtypeStruct((2, NC, N_PAD, HD), jnp.float32),
        mesh=mesh,
        scratch_types=[
            [pltpu.VMEM((LANES,), jnp.int32) for _ in range(NB)],
            [pltpu.VMEM((LANES,), jnp.int32) for _ in range(NB)],
            [pltpu.VMEM((LANES, HD), jnp.float32) for _ in range(NB)],
            pltpu.VMEM((NB, LANES), jnp.int32),
            pltpu.VMEM((NB, LANES), jnp.int32),
            pltpu.VMEM((LANES, HD), jnp.float32),
            pltpu.VMEM_SHARED((N_PAD, HD), jnp.float32),
            pltpu.SemaphoreType.DMA,
            [pltpu.SemaphoreType.DMA for _ in range(NB)],
            [pltpu.SemaphoreType.DMA for _ in range(NB)],
        ],
        compiler_params=pltpu.CompilerParams(use_tc_tiling_on_sc=False),
    )
    return deg_call, spmm_call


def _gelu(x):
    return x * 0.5 * (1.0 + lax.erf(x * 0.7071067811865476))


def _scale(deg):
    return jnp.where(deg > 0, lax.rsqrt(deg), 0.0)[:, 0:1]


def _pre_body(h_ref, ideg_ref, odeg_ref, wi_ref, bi_ref, wo_ref, bo_ref,
              u_ref, v_ref):
    h = h_ref[...]
    s_src = _scale(odeg_ref[...])
    s_dst = _scale(ideg_ref[...])
    h1 = jnp.dot(h, wi_ref[...], preferred_element_type=jnp.float32) + bi_ref[...]
    h2 = jnp.dot(h, wo_ref[...], preferred_element_type=jnp.float32) + bo_ref[...]
    u_ref[...] = s_src * h1
    v_ref[...] = s_dst * h2


def _post_body(a0_ref, a1_ref, ideg_ref, odeg_ref, wf1_ref, bf1_ref,
               wf2_ref, bf2_ref, o_ref):
    s_dst = _scale(ideg_ref[...])
    s_src = _scale(odeg_ref[...])
    x_in = _gelu(s_dst * a0_ref[...])
    x_out = _gelu(s_src * a1_ref[...])
    cat = jnp.concatenate([x_in, x_out], axis=1)
    z = _gelu(jnp.dot(cat, wf1_ref[...], preferred_element_type=jnp.float32)
              + bf1_ref[...])
    o_ref[...] = (jnp.dot(z, wf2_ref[...], preferred_element_type=jnp.float32)
                  + bf2_ref[...])


BN = 1000


def _row_spec(w):
    return pl.BlockSpec((BN, w), lambda i: (i, 0))


def _full_spec(r, c):
    return pl.BlockSpec((r, c), lambda i: (0, 0))


_pre_call = pl.pallas_call(
    _pre_body,
    grid=(N // BN,),
    in_specs=[
        _row_spec(D), _row_spec(16), _row_spec(16),
        _full_spec(D, D), _full_spec(1, D),
        _full_spec(D, D), _full_spec(1, D),
    ],
    out_specs=[_row_spec(D), _row_spec(D)],
    out_shape=[
        jax.ShapeDtypeStruct((N, D), jnp.float32),
        jax.ShapeDtypeStruct((N, D), jnp.float32),
    ],
)

_post_call = pl.pallas_call(
    _post_body,
    grid=(N // BN,),
    in_specs=[
        _row_spec(D), _row_spec(D), _row_spec(16), _row_spec(16),
        _full_spec(2 * D, D), _full_spec(1, D),
        _full_spec(D, D), _full_spec(1, D),
    ],
    out_specs=_row_spec(D),
    out_shape=jax.ShapeDtypeStruct((N, D), jnp.float32),
)


def kernel(x, edge_index, W_in0, b_in0, W_out0, b_out0, Wf1_0, bf1_0,
           Wf2_0, bf2_0, W_in1, b_in1, W_out1, b_out1, Wf1_1, bf1_1,
           Wf2_1, bf2_1):
    E = edge_index.shape[1]
    blk = NS * LANES * G
    e_pad = -(-E // blk) * blk
    pad = e_pad - E

    row = edge_index[0]
    col = edge_index[1]
    pad_src = jnp.broadcast_to(jnp.arange(pad, dtype=jnp.int32) % N, (NC, pad))
    pad_dst = jnp.broadcast_to(
        TRASH + (jnp.arange(pad, dtype=jnp.int32) % (N_PAD - N)), (NC, pad))
    src_p = jnp.concatenate(
        [jnp.stack([row, col + N]), pad_src],
        axis=1).reshape(NC, e_pad // LANES, LANES)
    dst_p = jnp.concatenate(
        [jnp.stack([col, row]), pad_dst],
        axis=1).reshape(NC, e_pad // LANES, LANES)

    ones16 = jnp.ones((LANES, 16), jnp.float32)
    zeros16 = jnp.zeros((ZROWS, 16), jnp.float32)
    zerosD = jnp.zeros((LANES, HD), jnp.float32)

    deg_call, spmm_call = _sc_calls()
    degs = deg_call(dst_p, ones16, zeros16)
    ideg = degs[0, :N]
    odeg = degs[1, :N]

    params = [
        (W_in0, b_in0, W_out0, b_out0, Wf1_0, bf1_0, Wf2_0, bf2_0),
        (W_in1, b_in1, W_out1, b_out1, Wf1_1, bf1_1, Wf2_1, bf2_1),
    ]
    h = x
    for (Wi, bi, Wo, bo, Wf1, bf1, Wf2, bf2) in params:
        u, v = _pre_call(h, ideg, odeg, Wi.T, bi.reshape(1, D),
                         Wo.T, bo.reshape(1, D))
        uv = jnp.concatenate([u, v], axis=0)
        agg = spmm_call(src_p, dst_p, uv[:, :HD], uv[:, HD:], zerosD)
        a0 = jnp.concatenate([agg[0, 0, :N], agg[1, 0, :N]], axis=1)
        a1 = jnp.concatenate([agg[0, 1, :N], agg[1, 1, :N]], axis=1)
        h = _post_call(a0, a1, ideg, odeg, Wf1.T, bf1.reshape(1, D),
                       Wf2.T, bf2.reshape(1, D))
    return h


# speedup vs baseline: 2.9731x; 1.0789x over previous
import functools

import jax
import jax.numpy as jnp
from jax import lax
from jax.experimental import pallas as pl
from jax.experimental.pallas import tpu as pltpu
from jax.experimental.pallas import tpu_sc as plsc

N = 10000
D = 128
NC = 2
NS = 16
LANES = 128
G = 4

N_PAD = 10240
TRASH = N
ZROWS = N_PAD // NS

def _deg_body(dst_hbm, ones_hbm, zeros_hbm, deg_out, idx_v, ones_v, z_v, acc):
    cid = lax.axis_index("c")
    sid = lax.axis_index("s")
    n_rows = dst_hbm.shape[1] // NS
    pltpu.sync_copy(ones_hbm, ones_v)
    pltpu.sync_copy(zeros_hbm, z_v)
    pltpu.sync_copy(z_v, acc.at[pl.ds(sid * ZROWS, ZROWS)])
    plsc.subcore_barrier()
    base = sid * n_rows

    def chunk(g, carry):
        pltpu.sync_copy(dst_hbm.at[cid, base + g], idx_v)
        pltpu.sync_copy(ones_v, acc.at[idx_v], add=True)
        return carry

    lax.fori_loop(0, n_rows, chunk, 0)
    plsc.subcore_barrier()
    pltpu.sync_copy(acc.at[pl.ds(sid * ZROWS, ZROWS)],
                    deg_out.at[cid, pl.ds(sid * ZROWS, ZROWS)])


HD = D // 2
NB = 8


def _vcopy_row(big, b, small):
    for w in range(LANES // 16):
        small[pl.ds(w * 16, 16)] = big[b, pl.ds(w * 16, 16)]


def _spmm_body(src_hbm, dst_hbm, uv0_hbm, uv1_hbm, zeros_hbm, agg_out,
               sidx_l, didx_l, rows_l, sbig, dbig, z_v, acc, isem,
               gsem_l, ssem_l):
    cid = lax.axis_index("c")
    sid = lax.axis_index("s")
    n_rows = src_hbm.shape[1] // NS
    base = sid * n_rows
    pltpu.sync_copy(zeros_hbm, z_v)

    n_supers = n_rows // NB
    for p, uv_hbm in enumerate((uv0_hbm, uv1_hbm)):
        for t in range(ZROWS // LANES):
            pltpu.sync_copy(z_v, acc.at[pl.ds(sid * ZROWS + t * LANES, LANES)])
        plsc.subcore_barrier()

        pltpu.sync_copy(src_hbm.at[cid, pl.ds(base, NB)], sbig)
        pltpu.sync_copy(dst_hbm.at[cid, pl.ds(base, NB)], dbig)
        for b in range(NB):
            _vcopy_row(sbig, b, sidx_l[b])
            _vcopy_row(dbig, b, didx_l[b])
            pltpu.async_copy(uv_hbm.at[sidx_l[b]], rows_l[b], gsem_l[b])

        def chunk(s, carry):
            g1 = base + jnp.minimum(s + 1, n_supers - 1) * NB
            icp0 = pltpu.async_copy(src_hbm.at[cid, pl.ds(g1, NB)], sbig, isem)
            icp1 = pltpu.async_copy(dst_hbm.at[cid, pl.ds(g1, NB)], dbig, isem)
            scps = []
            for b in range(NB):
                pltpu.make_async_copy(uv_hbm.at[sidx_l[b]], rows_l[b],
                                      gsem_l[b]).wait()
                scps.append(pltpu.async_copy(rows_l[b], acc.at[didx_l[b]],
                                             ssem_l[b], add=True))
            icp0.wait()
            icp1.wait()
            for b in range(NB):
                scps[b].wait()
                _vcopy_row(sbig, b, sidx_l[b])
                _vcopy_row(dbig, b, didx_l[b])
                pltpu.async_copy(uv_hbm.at[sidx_l[b]], rows_l[b], gsem_l[b])
            return carry

        lax.fori_loop(0, n_supers, chunk, 0)
        for b in range(NB):
            pltpu.make_async_copy(uv_hbm.at[sidx_l[b]], rows_l[b],
                                  gsem_l[b]).wait()
        plsc.subcore_barrier()
        pltpu.sync_copy(acc.at[pl.ds(sid * ZROWS, ZROWS)],
                        agg_out.at[p, cid, pl.ds(sid * ZROWS, ZROWS)])


@functools.cache
def _sc_calls():
    mesh = plsc.VectorSubcoreMesh(core_axis_name="c", subcore_axis_name="s")
    deg_call = pl.kernel(
        _deg_body,
        out_type=jax.ShapeDtypeStruct((NC, N_PAD, 16), jnp.float32),
        mesh=mesh,
        scratch_types=[
            pltpu.VMEM((LANES,), jnp.int32),
            pltpu.VMEM((LANES, 16), jnp.float32),
            pltpu.VMEM((ZROWS, 16), jnp.float32),
            pltpu.VMEM_SHARED((N_PAD, 16), jnp.float32),
        ],
        compiler_params=pltpu.CompilerParams(use_tc_tiling_on_sc=False),
    )
    spmm_call = pl.kernel(
        _spmm_body,
        out_type=jax.ShapeDtypeStruct((2, NC, N_PAD, HD), jnp.float32),
        mesh=mesh,
        scratch_types=[
            [pltpu.VMEM((LANES,), jnp.int32) for _ in range(NB)],
            [pltpu.VMEM((LANES,), jnp.int32) for _ in range(NB)],
            [pltpu.VMEM((LANES, HD), jnp.float32) for _ in range(NB)],
            pltpu.VMEM((NB, LANES), jnp.int32),
            pltpu.VMEM((NB, LANES), jnp.int32),
            pltpu.VMEM((LANES, HD), jnp.float32),
            pltpu.VMEM_SHARED((N_PAD, HD), jnp.float32),
            pltpu.SemaphoreType.DMA,
            [pltpu.SemaphoreType.DMA for _ in range(NB)],
            [pltpu.SemaphoreType.DMA for _ in range(NB)],
        ],
        compiler_params=pltpu.CompilerParams(use_tc_tiling_on_sc=False),
    )
    return deg_call, spmm_call


def _gelu(x):
    return x * 0.5 * (1.0 + lax.erf(x * 0.7071067811865476))


def _scale(deg):
    return jnp.where(deg > 0, lax.rsqrt(deg), 0.0)[:, 0:1]


def _pre_body(h_ref, ideg_ref, odeg_ref, wi_ref, bi_ref, wo_ref, bo_ref,
              uv0_ref, uv1_ref):
    h = h_ref[...]
    s_src = _scale(odeg_ref[0])
    s_dst = _scale(ideg_ref[0])
    h1 = jnp.dot(h, wi_ref[...], preferred_element_type=jnp.float32) + bi_ref[...]
    h2 = jnp.dot(h, wo_ref[...], preferred_element_type=jnp.float32) + bo_ref[...]
    u = s_src * h1
    v = s_dst * h2
    uv0_ref[0] = u[:, :HD]
    uv0_ref[1] = v[:, :HD]
    uv1_ref[0] = u[:, HD:]
    uv1_ref[1] = v[:, HD:]


def _post_body(a0_ref, a1_ref, ideg_ref, odeg_ref, wf1_ref, bf1_ref,
               wf2_ref, bf2_ref, o_ref):
    s_dst = _scale(ideg_ref[0])
    s_src = _scale(odeg_ref[0])
    x_in = _gelu(s_dst * jnp.concatenate([a0_ref[0, 0], a0_ref[1, 0]], axis=1))
    x_out = _gelu(s_src * jnp.concatenate([a1_ref[0, 0], a1_ref[1, 0]], axis=1))
    cat = jnp.concatenate([x_in, x_out], axis=1)
    z = _gelu(jnp.dot(cat, wf1_ref[...], preferred_element_type=jnp.float32)
              + bf1_ref[...])
    o_ref[...] = (jnp.dot(z, wf2_ref[...], preferred_element_type=jnp.float32)
                  + bf2_ref[...])


BN = 1000


def _row_spec(w):
    return pl.BlockSpec((BN, w), lambda i: (i, 0))


def _full_spec(r, c):
    return pl.BlockSpec((r, c), lambda i: (0, 0))


def _deg_spec(c):
    return pl.BlockSpec((1, BN, 16), lambda i, c=c: (c, i, 0))


def _agg_spec(c):
    return pl.BlockSpec((2, 1, BN, HD), lambda i, c=c: (0, c, i, 0))


_pre_call = pl.pallas_call(
    _pre_body,
    grid=(N // BN,),
    in_specs=[
        _row_spec(D), _deg_spec(0), _deg_spec(1),
        _full_spec(D, D), _full_spec(1, D),
        _full_spec(D, D), _full_spec(1, D),
    ],
    out_specs=[pl.BlockSpec((2, BN, HD), lambda i: (0, i, 0))] * 2,
    out_shape=[jax.ShapeDtypeStruct((2, N, HD), jnp.float32)] * 2,
)

_post_call = pl.pallas_call(
    _post_body,
    grid=(N // BN,),
    in_specs=[
        _agg_spec(0), _agg_spec(1), _deg_spec(0), _deg_spec(1),
        _full_spec(2 * D, D), _full_spec(1, D),
        _full_spec(D, D), _full_spec(1, D),
    ],
    out_specs=_row_spec(D),
    out_shape=jax.ShapeDtypeStruct((N, D), jnp.float32),
)


def kernel(x, edge_index, W_in0, b_in0, W_out0, b_out0, Wf1_0, bf1_0,
           Wf2_0, bf2_0, W_in1, b_in1, W_out1, b_out1, Wf1_1, bf1_1,
           Wf2_1, bf2_1):
    E = edge_index.shape[1]
    blk = NS * LANES * G
    e_pad = -(-E // blk) * blk
    pad = e_pad - E

    row = edge_index[0]
    col = edge_index[1]
    pad_src = jnp.broadcast_to(jnp.arange(pad, dtype=jnp.int32) % N, (NC, pad))
    pad_dst = jnp.broadcast_to(
        TRASH + (jnp.arange(pad, dtype=jnp.int32) % (N_PAD - N)), (NC, pad))
    src_p = jnp.concatenate(
        [jnp.stack([row, col + N]), pad_src],
        axis=1).reshape(NC, e_pad // LANES, LANES)
    dst_p = jnp.concatenate(
        [jnp.stack([col, row]), pad_dst],
        axis=1).reshape(NC, e_pad // LANES, LANES)

    ones16 = jnp.ones((LANES, 16), jnp.float32)
    zeros16 = jnp.zeros((ZROWS, 16), jnp.float32)
    zerosD = jnp.zeros((LANES, HD), jnp.float32)

    deg_call, spmm_call = _sc_calls()
    degs = deg_call(dst_p, ones16, zeros16)

    params = [
        (W_in0, b_in0, W_out0, b_out0, Wf1_0, bf1_0, Wf2_0, bf2_0),
        (W_in1, b_in1, W_out1, b_out1, Wf1_1, bf1_1, Wf2_1, bf2_1),
    ]
    h = x
    for (Wi, bi, Wo, bo, Wf1, bf1, Wf2, bf2) in params:
        uva, uvb = _pre_call(h, degs, degs, Wi.T, bi.reshape(1, D),
                             Wo.T, bo.reshape(1, D))
        agg = spmm_call(src_p, dst_p, uva.reshape(2 * N, HD),
                        uvb.reshape(2 * N, HD), zerosD)
        h = _post_call(agg, agg, degs, degs, Wf1.T, bf1.reshape(1, D),
                       Wf2.T, bf2.reshape(1, D))
    return h


# speedup vs baseline: 3.3586x; 1.1297x over previous
import functools

import jax
import jax.numpy as jnp
from jax import lax
from jax.experimental import pallas as pl
from jax.experimental.pallas import tpu as pltpu
from jax.experimental.pallas import tpu_sc as plsc

N = 10000
D = 128
NC = 2
NS = 16
LANES = 128
G = 4

N_PAD = 10240
TRASH = N
ZROWS = N_PAD // NS

def _deg_body(dst_hbm, ones_hbm, zeros_hbm, deg_out, didx_l, dbig, ones_v,
              z_v, acc, isem, ssem_l):
    cid = lax.axis_index("c")
    sid = lax.axis_index("s")
    n_rows = dst_hbm.shape[1] // NS
    n_supers = n_rows // NB
    pltpu.sync_copy(ones_hbm, ones_v)
    pltpu.sync_copy(zeros_hbm, z_v)
    pltpu.sync_copy(z_v, acc.at[pl.ds(sid * ZROWS, ZROWS)])
    plsc.subcore_barrier()
    base = sid * n_rows
    pltpu.sync_copy(dst_hbm.at[cid, pl.ds(base, NB)], dbig)
    for b in range(NB):
        _vcopy_row(dbig, b, didx_l[b])

    def chunk(s, carry):
        g1 = base + jnp.minimum(s + 1, n_supers - 1) * NB
        icp = pltpu.async_copy(dst_hbm.at[cid, pl.ds(g1, NB)], dbig, isem)
        scps = [pltpu.async_copy(ones_v, acc.at[didx_l[b]], ssem_l[b],
                                 add=True) for b in range(NB)]
        icp.wait()
        for b in range(NB):
            scps[b].wait()
            _vcopy_row(dbig, b, didx_l[b])
        return carry

    lax.fori_loop(0, n_supers, chunk, 0)
    plsc.subcore_barrier()
    pltpu.sync_copy(acc.at[pl.ds(sid * ZROWS, ZROWS)],
                    deg_out.at[cid, pl.ds(sid * ZROWS, ZROWS)])


HD = D // 2
NB = 8


def _vcopy_row(big, b, small):
    for w in range(LANES // 16):
        small[pl.ds(w * 16, 16)] = big[b, pl.ds(w * 16, 16)]


def _spmm_body(src_hbm, dst_hbm, uv0_hbm, uv1_hbm, zeros_hbm, agg_out,
               sidx_l, didx_l, rows_l, sbig, dbig, z_v, acc, isem,
               gsem_l, ssem_l):
    cid = lax.axis_index("c")
    sid = lax.axis_index("s")
    n_rows = src_hbm.shape[1] // NS
    base = sid * n_rows
    pltpu.sync_copy(zeros_hbm, z_v)

    n_supers = n_rows // NB
    for p, uv_hbm in enumerate((uv0_hbm, uv1_hbm)):
        for t in range(ZROWS // LANES):
            pltpu.sync_copy(z_v, acc.at[pl.ds(sid * ZROWS + t * LANES, LANES)])
        plsc.subcore_barrier()

        pltpu.sync_copy(src_hbm.at[cid, pl.ds(base, NB)], sbig)
        pltpu.sync_copy(dst_hbm.at[cid, pl.ds(base, NB)], dbig)
        for b in range(NB):
            _vcopy_row(sbig, b, sidx_l[b])
            _vcopy_row(dbig, b, didx_l[b])
            pltpu.async_copy(uv_hbm.at[sidx_l[b]], rows_l[b], gsem_l[b])

        def chunk(s, carry):
            g1 = base + jnp.minimum(s + 1, n_supers - 1) * NB
            icp0 = pltpu.async_copy(src_hbm.at[cid, pl.ds(g1, NB)], sbig, isem)
            icp1 = pltpu.async_copy(dst_hbm.at[cid, pl.ds(g1, NB)], dbig, isem)
            scps = []
            for b in range(NB):
                pltpu.make_async_copy(uv_hbm.at[sidx_l[b]], rows_l[b],
                                      gsem_l[b]).wait()
                scps.append(pltpu.async_copy(rows_l[b], acc.at[didx_l[b]],
                                             ssem_l[b], add=True))
            icp0.wait()
            icp1.wait()
            for b in range(NB):
                scps[b].wait()
                _vcopy_row(sbig, b, sidx_l[b])
                _vcopy_row(dbig, b, didx_l[b])
                pltpu.async_copy(uv_hbm.at[sidx_l[b]], rows_l[b], gsem_l[b])
            return carry

        lax.fori_loop(0, n_supers, chunk, 0)
        for b in range(NB):
            pltpu.make_async_copy(uv_hbm.at[sidx_l[b]], rows_l[b],
                                  gsem_l[b]).wait()
        plsc.subcore_barrier()
        pltpu.sync_copy(acc.at[pl.ds(sid * ZROWS, ZROWS)],
                        agg_out.at[p, cid, pl.ds(sid * ZROWS, ZROWS)])


@functools.cache
def _sc_calls():
    mesh = plsc.VectorSubcoreMesh(core_axis_name="c", subcore_axis_name="s")
    deg_call = pl.kernel(
        _deg_body,
        out_type=jax.ShapeDtypeStruct((NC, N_PAD, 16), jnp.float32),
        mesh=mesh,
        scratch_types=[
            [pltpu.VMEM((LANES,), jnp.int32) for _ in range(NB)],
            pltpu.VMEM((NB, LANES), jnp.int32),
            pltpu.VMEM((LANES, 16), jnp.float32),
            pltpu.VMEM((ZROWS, 16), jnp.float32),
            pltpu.VMEM_SHARED((N_PAD, 16), jnp.float32),
            pltpu.SemaphoreType.DMA,
            [pltpu.SemaphoreType.DMA for _ in range(NB)],
        ],
        compiler_params=pltpu.CompilerParams(use_tc_tiling_on_sc=False),
    )
    spmm_call = pl.kernel(
        _spmm_body,
        out_type=jax.ShapeDtypeStruct((2, NC, N_PAD, HD), jnp.float32),
        mesh=mesh,
        scratch_types=[
            [pltpu.VMEM((LANES,), jnp.int32) for _ in range(NB)],
            [pltpu.VMEM((LANES,), jnp.int32) for _ in range(NB)],
            [pltpu.VMEM((LANES, HD), jnp.float32) for _ in range(NB)],
            pltpu.VMEM((NB, LANES), jnp.int32),
            pltpu.VMEM((NB, LANES), jnp.int32),
            pltpu.VMEM((LANES, HD), jnp.float32),
            pltpu.VMEM_SHARED((N_PAD, HD), jnp.float32),
            pltpu.SemaphoreType.DMA,
            [pltpu.SemaphoreType.DMA for _ in range(NB)],
            [pltpu.SemaphoreType.DMA for _ in range(NB)],
        ],
        compiler_params=pltpu.CompilerParams(use_tc_tiling_on_sc=False),
    )
    return deg_call, spmm_call


def _gelu(x):
    return x * 0.5 * (1.0 + lax.erf(x * 0.7071067811865476))


def _scale(deg):
    return jnp.where(deg > 0, lax.rsqrt(deg), 0.0)[:, 0:1]


def _pre_body(h_ref, ideg_ref, odeg_ref, wi_ref, bi_ref, wo_ref, bo_ref,
              uv0_ref, uv1_ref):
    h = h_ref[...]
    s_src = _scale(odeg_ref[0])
    s_dst = _scale(ideg_ref[0])
    h1 = jnp.dot(h, wi_ref[...], preferred_element_type=jnp.float32) + bi_ref[...]
    h2 = jnp.dot(h, wo_ref[...], preferred_element_type=jnp.float32) + bo_ref[...]
    u = s_src * h1
    v = s_dst * h2
    uv0_ref[0] = u[:, :HD]
    uv0_ref[1] = v[:, :HD]
    uv1_ref[0] = u[:, HD:]
    uv1_ref[1] = v[:, HD:]


def _post_body(a0_ref, a1_ref, ideg_ref, odeg_ref, wf1_ref, bf1_ref,
               wf2_ref, bf2_ref, o_ref):
    s_dst = _scale(ideg_ref[0])
    s_src = _scale(odeg_ref[0])
    x_in = _gelu(s_dst * jnp.concatenate([a0_ref[0, 0], a0_ref[1, 0]], axis=1))
    x_out = _gelu(s_src * jnp.concatenate([a1_ref[0, 0], a1_ref[1, 0]], axis=1))
    cat = jnp.concatenate([x_in, x_out], axis=1)
    z = _gelu(jnp.dot(cat, wf1_ref[...], preferred_element_type=jnp.float32)
              + bf1_ref[...])
    o_ref[...] = (jnp.dot(z, wf2_ref[...], preferred_element_type=jnp.float32)
                  + bf2_ref[...])


BN = 1000


def _row_spec(w):
    return pl.BlockSpec((BN, w), lambda i: (i, 0))


def _full_spec(r, c):
    return pl.BlockSpec((r, c), lambda i: (0, 0))


def _deg_spec(c):
    return pl.BlockSpec((1, BN, 16), lambda i, c=c: (c, i, 0))


def _agg_spec(c):
    return pl.BlockSpec((2, 1, BN, HD), lambda i, c=c: (0, c, i, 0))


_pre_call = pl.pallas_call(
    _pre_body,
    grid=(N // BN,),
    in_specs=[
        _row_spec(D), _deg_spec(0), _deg_spec(1),
        _full_spec(D, D), _full_spec(1, D),
        _full_spec(D, D), _full_spec(1, D),
    ],
    out_specs=[pl.BlockSpec((2, BN, HD), lambda i: (0, i, 0))] * 2,
    out_shape=[jax.ShapeDtypeStruct((2, N, HD), jnp.float32)] * 2,
)

_post_call = pl.pallas_call(
    _post_body,
    grid=(N // BN,),
    in_specs=[
        _agg_spec(0), _agg_spec(1), _deg_spec(0), _deg_spec(1),
        _full_spec(2 * D, D), _full_spec(1, D),
        _full_spec(D, D), _full_spec(1, D),
    ],
    out_specs=_row_spec(D),
    out_shape=jax.ShapeDtypeStruct((N, D), jnp.float32),
)


def kernel(x, edge_index, W_in0, b_in0, W_out0, b_out0, Wf1_0, bf1_0,
           Wf2_0, bf2_0, W_in1, b_in1, W_out1, b_out1, Wf1_1, bf1_1,
           Wf2_1, bf2_1):
    E = edge_index.shape[1]
    blk = NS * LANES * G
    e_pad = -(-E // blk) * blk
    pad = e_pad - E

    row = edge_index[0]
    col = edge_index[1]
    pad_src = jnp.broadcast_to(jnp.arange(pad, dtype=jnp.int32) % N, (NC, pad))
    pad_dst = jnp.broadcast_to(
        TRASH + (jnp.arange(pad, dtype=jnp.int32) % (N_PAD - N)), (NC, pad))
    src_p = jnp.concatenate(
        [jnp.stack([row, col + N]), pad_src],
        axis=1).reshape(NC, e_pad // LANES, LANES)
    dst_p = jnp.concatenate(
        [jnp.stack([col, row]), pad_dst],
        axis=1).reshape(NC, e_pad // LANES, LANES)

    ones16 = jnp.ones((LANES, 16), jnp.float32)
    zeros16 = jnp.zeros((ZROWS, 16), jnp.float32)
    zerosD = jnp.zeros((LANES, HD), jnp.float32)

    deg_call, spmm_call = _sc_calls()
    degs = deg_call(dst_p, ones16, zeros16)

    params = [
        (W_in0, b_in0, W_out0, b_out0, Wf1_0, bf1_0, Wf2_0, bf2_0),
        (W_in1, b_in1, W_out1, b_out1, Wf1_1, bf1_1, Wf2_1, bf2_1),
    ]
    h = x
    for (Wi, bi, Wo, bo, Wf1, bf1, Wf2, bf2) in params:
        uva, uvb = _pre_call(h, degs, degs, Wi.T, bi.reshape(1, D),
                             Wo.T, bo.reshape(1, D))
        agg = spmm_call(src_p, dst_p, uva.reshape(2 * N, HD),
                        uvb.reshape(2 * N, HD), zerosD)
        h = _post_call(agg, agg, degs, degs, Wf1.T, bf1.reshape(1, D),
                       Wf2.T, bf2.reshape(1, D))
    return h
